# baseline (speedup 1.0000x reference)
import functools

import jax
import jax.numpy as jnp
from jax import lax
from jax.experimental import pallas as pl
from jax.experimental.pallas import tpu as pltpu
from jax.experimental.pallas import tpu_sc as plsc

N = 10000
E = 320000
D = 128

NC = 2
NS = 16
NW = NC * NS
EP = E // NW
CH = 80
NCHUNK = EP // CH
NRCH = N // CH

RB = 2000


def _artanh(x):
    return 0.5 * (jnp.log1p(x) - jnp.log1p(-x))


def _pre_body(c_ref, x_ref, wt_ref, b_ref, o_ref):
    c = jnp.abs(c_ref[0])
    sqrt_c = jnp.sqrt(c)
    x = x_ref[...]
    norm = jnp.sqrt(jnp.sum(x * x, axis=1, keepdims=True))
    norm = jnp.clip(norm, 1e-15, None)
    arg = jnp.clip(sqrt_c * norm, -1.0 + 1e-5, 1.0 - 1e-5)
    tan = _artanh(arg) * x / (sqrt_c * norm)
    o_ref[...] = (
        jnp.dot(tan, wt_ref[...], preferred_element_type=jnp.float32)
        + b_ref[...]
    )


def _pre(emb, wt, b2, cval):
    return pl.pallas_call(
        _pre_body,
        grid=(N // RB,),
        in_specs=[
            pl.BlockSpec(memory_space=pltpu.SMEM),
            pl.BlockSpec((RB, D), lambda i: (i, 0)),
            pl.BlockSpec((D, D), lambda i: (0, 0)),
            pl.BlockSpec((1, D), lambda i: (0, 0)),
        ],
        out_specs=pl.BlockSpec((RB, D), lambda i: (i, 0)),
        out_shape=jax.ShapeDtypeStruct((N, D), jnp.float32),
    )(cval, emb, wt, b2)


def _sc_body(trans_hbm, src_hbm, dst_hbm, sums_hbm, cnt_hbm,
             src_all, dst_all, rows2, ones_v, acc_sh, cnt_sh,
             gsem0, gsem1, ssem0, ssem1):
    cid = lax.axis_index("c")
    sid = lax.axis_index("s")
    wid = cid * NS + sid

    pltpu.sync_copy(src_hbm.at[pl.ds(wid * EP, EP)], src_all)
    pltpu.sync_copy(dst_hbm.at[wid], dst_all)

    z16 = jnp.zeros((16,), jnp.float32)

    def zr(i, _):
        def zc(j, _):
            rows2[0, i, pl.ds(j * 16, 16)] = z16
            return 0
        return lax.fori_loop(0, D // 16, zc, 0)
    lax.fori_loop(0, CH, zr, 0)

    def zo(i, _):
        ones_v[pl.ds(i * 16, 16)] = z16
        return 0
    lax.fori_loop(0, CH // 16, zo, 0)

    nmine = (NRCH - sid + NS - 1) // NS

    def zacc(k, _):
        r = (sid + k * NS) * CH
        pltpu.sync_copy(rows2.at[0], acc_sh.at[pl.ds(r, CH)])
        pltpu.sync_copy(ones_v, cnt_sh.at[pl.ds(r, CH)])
        return 0
    lax.fori_loop(0, nmine, zacc, 0)

    o16 = jnp.ones((16,), jnp.float32)
    def fo(i, _):
        ones_v[pl.ds(i * 16, 16)] = o16
        return 0
    lax.fori_loop(0, CH // 16, fo, 0)

    plsc.subcore_barrier()

    def gslice(a):
        return src_all.at[pl.ds(a * CH, CH)]

    pltpu.async_copy(trans_hbm.at[gslice(0)], rows2.at[0], gsem0)

    def pipe(p, _):
        a = 2 * p
        pltpu.make_async_copy(
            trans_hbm.at[gslice(a)], rows2.at[0], gsem0).wait()
        pltpu.async_copy(trans_hbm.at[gslice(a + 1)], rows2.at[1], gsem1)
        pltpu.async_copy(rows2.at[0], acc_sh.at[dst_all.at[a]], ssem0,
                         add=True)
        pltpu.async_copy(ones_v, cnt_sh.at[dst_all.at[a]], ssem0, add=True)
        pltpu.make_async_copy(
            trans_hbm.at[gslice(a + 1)], rows2.at[1], gsem1).wait()
        pltpu.make_async_copy(
            rows2.at[0], acc_sh.at[dst_all.at[a]], ssem0).wait()
        pltpu.make_async_copy(
            ones_v, cnt_sh.at[dst_all.at[a]], ssem0).wait()
        pltpu.async_copy(trans_hbm.at[gslice(a + 2)], rows2.at[0], gsem0)
        pltpu.async_copy(rows2.at[1], acc_sh.at[dst_all.at[a + 1]], ssem1,
                         add=True)
        pltpu.async_copy(ones_v, cnt_sh.at[dst_all.at[a + 1]], ssem1,
                         add=True)
        pltpu.make_async_copy(
            rows2.at[1], acc_sh.at[dst_all.at[a + 1]], ssem1).wait()
        pltpu.make_async_copy(
            ones_v, cnt_sh.at[dst_all.at[a + 1]], ssem1).wait()
        return 0
    lax.fori_loop(0, (NCHUNK - 1) // 2, pipe, 0)

    last = NCHUNK - 1
    pltpu.make_async_copy(
        trans_hbm.at[gslice(last)], rows2.at[0], gsem0).wait()
    pltpu.sync_copy(rows2.at[0], acc_sh.at[dst_all.at[last]], add=True)
    pltpu.sync_copy(ones_v, cnt_sh.at[dst_all.at[last]], add=True)

    plsc.subcore_barrier()

    def wacc(k, _):
        r = (sid + k * NS) * CH
        pltpu.sync_copy(acc_sh.at[pl.ds(r, CH)], sums_hbm.at[cid, pl.ds(r, CH)])
        pltpu.sync_copy(cnt_sh.at[pl.ds(r, CH)], ones_v)
        pltpu.sync_copy(ones_v, cnt_hbm.at[pl.ds(cid * N + r, CH)])
        return 0
    lax.fori_loop(0, nmine, wacc, 0)


_sc_agg = pl.kernel(
    _sc_body,
    out_type=[
        jax.ShapeDtypeStruct((NC, N, D), jnp.float32),
        jax.ShapeDtypeStruct((NC * N,), jnp.float32),
    ],
    mesh=plsc.VectorSubcoreMesh(
        core_axis_name="c", subcore_axis_name="s",
        num_cores=NC, num_subcores=NS),
    scratch_types=[
        pltpu.VMEM((EP,), jnp.int32),
        pltpu.VMEM((NCHUNK, CH), jnp.int32),
        pltpu.VMEM((2, CH, D), jnp.float32),
        pltpu.VMEM((CH,), jnp.float32),
        pltpu.VMEM_SHARED((N, D), jnp.float32),
        pltpu.VMEM_SHARED((N,), jnp.float32),
        pltpu.SemaphoreType.DMA,
        pltpu.SemaphoreType.DMA,
        pltpu.SemaphoreType.DMA,
        pltpu.SemaphoreType.DMA,
    ],
)


RBB = 2048
NPAD = 10240


def _post_body(c_ref, s_ref, n_ref, o_ref):
    c = jnp.abs(c_ref[0])
    sqrt_c = jnp.sqrt(c)
    s = s_ref[0] + s_ref[1]
    i = pl.program_id(0)
    cnt = (n_ref[0, pl.ds(i * RBB, RBB)] + n_ref[1, pl.ds(i * RBB, RBB)])[:, None]
    neigh = jnp.where(cnt > 0, s / jnp.clip(cnt, 1.0, None), 0.0)
    norm = jnp.sqrt(jnp.sum(neigh * neigh, axis=1, keepdims=True))
    norm = jnp.clip(norm, 1e-15, None)
    o_ref[...] = jnp.tanh(sqrt_c * norm) * neigh / (sqrt_c * norm)


def _post(sums, cnts, cval):
    cnts_p = jnp.concatenate(
        [cnts, jnp.zeros((NC, NPAD - N), jnp.float32)], axis=1)
    return pl.pallas_call(
        _post_body,
        grid=((N + RBB - 1) // RBB,),
        in_specs=[
            pl.BlockSpec(memory_space=pltpu.SMEM),
            pl.BlockSpec((NC, RBB, D), lambda i: (0, i, 0)),
            pl.BlockSpec((NC, NPAD), lambda i: (0, 0)),
        ],
        out_specs=pl.BlockSpec((RBB, D), lambda i: (i, 0)),
        out_shape=jax.ShapeDtypeStruct((N, D), jnp.float32),
    )(cval, sums, cnts_p)


def kernel(edge_index, node_embeddings, W, b, curvature):
    cval = jnp.abs(curvature).astype(jnp.float32)
    wt = W.T
    b2 = b.reshape(1, D)
    transformed = _pre(node_embeddings, wt, b2, cval)
    dst3 = edge_index[1].reshape(NW, NCHUNK, CH)
    sums, cnts = _sc_agg(transformed, edge_index[0], dst3)
    return _post(sums, cnts.reshape(NC, N), cval)


# speedup vs baseline: 1.1914x; 1.1914x over previous
import functools

import jax
import jax.numpy as jnp
from jax import lax
from jax.experimental import pallas as pl
from jax.experimental.pallas import tpu as pltpu
from jax.experimental.pallas import tpu_sc as plsc

N = 10000
E = 320000
D = 128

NC = 2
NS = 16
NW = NC * NS
EP = E // NW
CH = 80
NCHUNK = EP // CH
NRCH = N // CH

RB = 2000


def _artanh(x):
    return 0.5 * (jnp.log1p(x) - jnp.log1p(-x))


def _pre_body(c_ref, x_ref, wt_ref, b_ref, o_ref):
    c = jnp.abs(c_ref[0])
    sqrt_c = jnp.sqrt(c)
    x = x_ref[...]
    norm = jnp.sqrt(jnp.sum(x * x, axis=1, keepdims=True))
    norm = jnp.clip(norm, 1e-15, None)
    arg = jnp.clip(sqrt_c * norm, -1.0 + 1e-5, 1.0 - 1e-5)
    tan = _artanh(arg) * x / (sqrt_c * norm)
    o_ref[...] = (
        jnp.dot(tan, wt_ref[...], preferred_element_type=jnp.float32)
        + b_ref[...]
    )


def _pre(emb, wt, b2, cval):
    return pl.pallas_call(
        _pre_body,
        grid=(N // RB,),
        in_specs=[
            pl.BlockSpec(memory_space=pltpu.SMEM),
            pl.BlockSpec((RB, D), lambda i: (i, 0)),
            pl.BlockSpec((D, D), lambda i: (0, 0)),
            pl.BlockSpec((1, D), lambda i: (0, 0)),
        ],
        out_specs=pl.BlockSpec((RB, D), lambda i: (i, 0)),
        out_shape=jax.ShapeDtypeStruct((N, D), jnp.float32),
    )(cval, emb, wt, b2)


def _sc_body(trans_hbm, src_hbm, dst_hbm, sums_hbm, cnt_hbm,
             src_all, dst_all, rows2, ones_v, acc_sh, cnt_sh,
             gsem0, gsem1, ssem0, ssem1):
    cid = lax.axis_index("c")
    sid = lax.axis_index("s")
    wid = cid * NS + sid

    pltpu.sync_copy(src_hbm.at[pl.ds(wid * EP, EP)], src_all)
    pltpu.sync_copy(dst_hbm.at[wid], dst_all)

    z16 = jnp.zeros((16,), jnp.float32)

    def zr(i, _):
        def zc(j, _):
            rows2[0, i, pl.ds(j * 16, 16)] = z16
            return 0
        return lax.fori_loop(0, D // 16, zc, 0)
    lax.fori_loop(0, CH, zr, 0)

    def zo(i, _):
        ones_v[pl.ds(i * 16, 16)] = z16
        return 0
    lax.fori_loop(0, CH // 16, zo, 0)

    nmine = (NRCH - sid + NS - 1) // NS

    def zacc(k, _):
        r = (sid + k * NS) * CH
        pltpu.sync_copy(rows2.at[0], acc_sh.at[pl.ds(r, CH)])
        pltpu.sync_copy(ones_v, cnt_sh.at[pl.ds(r, CH)])
        return 0
    lax.fori_loop(0, nmine, zacc, 0)

    o16 = jnp.ones((16,), jnp.float32)
    def fo(i, _):
        ones_v[pl.ds(i * 16, 16)] = o16
        return 0
    lax.fori_loop(0, CH // 16, fo, 0)

    plsc.subcore_barrier()

    def gslice(a):
        return src_all.at[pl.ds(a * CH, CH)]

    def g_wait(a, buf, sem):
        pltpu.make_async_copy(trans_hbm.at[gslice(a)], rows2.at[buf], sem).wait()

    def s_issue(a, buf, sem):
        pltpu.async_copy(rows2.at[buf], acc_sh.at[dst_all.at[a]], sem,
                         add=True)
        pltpu.async_copy(ones_v, cnt_sh.at[dst_all.at[a]], sem, add=True)

    def s_wait(a, buf, sem):
        pltpu.make_async_copy(rows2.at[buf], acc_sh.at[dst_all.at[a]],
                              sem).wait()
        pltpu.make_async_copy(ones_v, cnt_sh.at[dst_all.at[a]], sem).wait()

    pltpu.async_copy(trans_hbm.at[gslice(0)], rows2.at[0], gsem0)
    g_wait(0, 0, gsem0)
    pltpu.async_copy(trans_hbm.at[gslice(1)], rows2.at[1], gsem1)
    s_issue(0, 0, ssem0)
    g_wait(1, 1, gsem1)
    s_wait(0, 0, ssem0)
    pltpu.async_copy(trans_hbm.at[gslice(2)], rows2.at[0], gsem0)
    s_issue(1, 1, ssem1)

    def pipe(p, _):
        a = 2 * p
        s_wait(a - 1, 1, ssem1)
        pltpu.async_copy(trans_hbm.at[gslice(a + 1)], rows2.at[1], gsem1)
        g_wait(a, 0, gsem0)
        s_issue(a, 0, ssem0)
        g_wait(a + 1, 1, gsem1)
        s_wait(a, 0, ssem0)
        pltpu.async_copy(trans_hbm.at[gslice(a + 2)], rows2.at[0], gsem0)
        s_issue(a + 1, 1, ssem1)
        return 0
    lax.fori_loop(1, (NCHUNK - 1) // 2, pipe, 0)

    last = NCHUNK - 1
    s_wait(last - 1, 1, ssem1)
    g_wait(last, 0, gsem0)
    pltpu.sync_copy(rows2.at[0], acc_sh.at[dst_all.at[last]], add=True)
    pltpu.sync_copy(ones_v, cnt_sh.at[dst_all.at[last]], add=True)

    plsc.subcore_barrier()

    def wacc(k, _):
        r = (sid + k * NS) * CH
        pltpu.sync_copy(acc_sh.at[pl.ds(r, CH)], sums_hbm.at[cid, pl.ds(r, CH)])
        pltpu.sync_copy(cnt_sh.at[pl.ds(r, CH)], ones_v)
        pltpu.sync_copy(ones_v, cnt_hbm.at[pl.ds(cid * N + r, CH)])
        return 0
    lax.fori_loop(0, nmine, wacc, 0)


_sc_agg = pl.kernel(
    _sc_body,
    out_type=[
        jax.ShapeDtypeStruct((NC, N, D), jnp.float32),
        jax.ShapeDtypeStruct((NC * N,), jnp.float32),
    ],
    mesh=plsc.VectorSubcoreMesh(
        core_axis_name="c", subcore_axis_name="s",
        num_cores=NC, num_subcores=NS),
    scratch_types=[
        pltpu.VMEM((EP,), jnp.int32),
        pltpu.VMEM((NCHUNK, CH), jnp.int32),
        pltpu.VMEM((2, CH, D), jnp.float32),
        pltpu.VMEM((CH,), jnp.float32),
        pltpu.VMEM_SHARED((N, D), jnp.float32),
        pltpu.VMEM_SHARED((N,), jnp.float32),
        pltpu.SemaphoreType.DMA,
        pltpu.SemaphoreType.DMA,
        pltpu.SemaphoreType.DMA,
        pltpu.SemaphoreType.DMA,
    ],
)


RBB = 2048
NPAD = 10240


def _post_body(c_ref, s_ref, n_ref, o_ref):
    c = jnp.abs(c_ref[0])
    sqrt_c = jnp.sqrt(c)
    s = s_ref[0] + s_ref[1]
    i = pl.program_id(0)
    cnt = (n_ref[0, pl.ds(i * RBB, RBB)] + n_ref[1, pl.ds(i * RBB, RBB)])[:, None]
    neigh = jnp.where(cnt > 0, s / jnp.clip(cnt, 1.0, None), 0.0)
    norm = jnp.sqrt(jnp.sum(neigh * neigh, axis=1, keepdims=True))
    norm = jnp.clip(norm, 1e-15, None)
    o_ref[...] = jnp.tanh(sqrt_c * norm) * neigh / (sqrt_c * norm)


def _post(sums, cnts, cval):
    cnts_p = jnp.concatenate(
        [cnts, jnp.zeros((NC, NPAD - N), jnp.float32)], axis=1)
    return pl.pallas_call(
        _post_body,
        grid=((N + RBB - 1) // RBB,),
        in_specs=[
            pl.BlockSpec(memory_space=pltpu.SMEM),
            pl.BlockSpec((NC, RBB, D), lambda i: (0, i, 0)),
            pl.BlockSpec((NC, NPAD), lambda i: (0, 0)),
        ],
        out_specs=pl.BlockSpec((RBB, D), lambda i: (i, 0)),
        out_shape=jax.ShapeDtypeStruct((N, D), jnp.float32),
    )(cval, sums, cnts_p)


def kernel(edge_index, node_embeddings, W, b, curvature):
    cval = jnp.abs(curvature).astype(jnp.float32)
    wt = W.T
    b2 = b.reshape(1, D)
    transformed = _pre(node_embeddings, wt, b2, cval)
    dst3 = edge_index[1].reshape(NW, NCHUNK, CH)
    sums, cnts = _sc_agg(transformed, edge_index[0], dst3)
    return _post(sums, cnts.reshape(NC, N), cval)


# speedup vs baseline: 1.2626x; 1.0598x over previous
import functools

import jax
import jax.numpy as jnp
from jax import lax
from jax.experimental import pallas as pl
from jax.experimental.pallas import tpu as pltpu
from jax.experimental.pallas import tpu_sc as plsc

N = 10000
E = 320000
D = 128

NC = 2
NS = 16
NW = NC * NS
EP = E // NW
CH = 80
NCHUNK = EP // CH
NRCH = N // CH

RB = 2000


def _artanh(x):
    return 0.5 * (jnp.log1p(x) - jnp.log1p(-x))


def _pre_body(c_ref, x_ref, wt_ref, b_ref, o_ref):
    c = jnp.abs(c_ref[0])
    sqrt_c = jnp.sqrt(c)
    x = x_ref[...]
    norm = jnp.sqrt(jnp.sum(x * x, axis=1, keepdims=True))
    norm = jnp.clip(norm, 1e-15, None)
    arg = jnp.clip(sqrt_c * norm, -1.0 + 1e-5, 1.0 - 1e-5)
    tan = _artanh(arg) * x / (sqrt_c * norm)
    o_ref[...] = (
        jnp.dot(tan, wt_ref[...], preferred_element_type=jnp.float32)
        + b_ref[...]
    )


def _pre(emb, wt, b2, cval):
    return pl.pallas_call(
        _pre_body,
        grid=(N // RB,),
        in_specs=[
            pl.BlockSpec(memory_space=pltpu.SMEM),
            pl.BlockSpec((RB, D), lambda i: (i, 0)),
            pl.BlockSpec((D, D), lambda i: (0, 0)),
            pl.BlockSpec((1, D), lambda i: (0, 0)),
        ],
        out_specs=pl.BlockSpec((RB, D), lambda i: (i, 0)),
        out_shape=jax.ShapeDtypeStruct((N, D), jnp.float32),
    )(cval, emb, wt, b2)


def _sc_body(trans_hbm, src_hbm, dst_hbm, sums_hbm, cnt_hbm,
             src_all, stage, rows2, ones_v, acc_sh, cnt_sh,
             gsem0, gsem1, gsem2, ssem0, ssem1, ssem2):
    cid = lax.axis_index("c")
    sid = lax.axis_index("s")
    wid = cid * NS + sid

    pltpu.sync_copy(src_hbm.at[pl.ds(wid * EP, EP)], src_all)

    z16 = jnp.zeros((16,), jnp.float32)

    def zr(i, _):
        def zc(j, _):
            rows2[0, i, pl.ds(j * 16, 16)] = z16
            return 0
        return lax.fori_loop(0, D // 16, zc, 0)
    lax.fori_loop(0, CH, zr, 0)

    def zo(i, _):
        ones_v[pl.ds(i * 16, 16)] = z16
        return 0
    lax.fori_loop(0, CH // 16, zo, 0)

    nmine = (NRCH - sid + NS - 1) // NS

    def zacc(k, _):
        r = (sid + k * NS) * CH
        pltpu.sync_copy(rows2.at[0], acc_sh.at[pl.ds(r, CH)])
        pltpu.sync_copy(ones_v, cnt_sh.at[pl.ds(r, CH)])
        return 0
    lax.fori_loop(0, nmine, zacc, 0)

    o16 = jnp.ones((16,), jnp.float32)
    def fo(i, _):
        ones_v[pl.ds(i * 16, 16)] = o16
        return 0
    lax.fori_loop(0, CH // 16, fo, 0)

    plsc.subcore_barrier()

    def gslice(a):
        return src_all.at[pl.ds(a * CH, CH)]

    def dslice(a):
        return dst_hbm.at[pl.ds(wid * EP + a * CH, CH)]

    gsems = (gsem0, gsem1, gsem2)
    ssems = (ssem0, ssem1, ssem2)

    def g_issue(a, b):
        pltpu.async_copy(trans_hbm.at[gslice(a)], rows2.at[b], gsems[b])
        pltpu.async_copy(dslice(a), stage.at[b], gsems[b])

    def g_wait(a, b):
        pltpu.make_async_copy(
            trans_hbm.at[gslice(a)], rows2.at[b], gsems[b]).wait()
        pltpu.make_async_copy(dslice(a), stage.at[b], gsems[b]).wait()

    def s_issue(a, b):
        pltpu.async_copy(rows2.at[b], acc_sh.at[stage.at[b]], ssems[b],
                         add=True)
        pltpu.async_copy(ones_v, cnt_sh.at[stage.at[b]], ssems[b], add=True)

    def s_wait(a, b):
        pltpu.make_async_copy(rows2.at[b], acc_sh.at[stage.at[b]],
                              ssems[b]).wait()
        pltpu.make_async_copy(ones_v, cnt_sh.at[stage.at[b]],
                              ssems[b]).wait()

    g_issue(0, 0)
    g_issue(1, 1)
    g_wait(0, 0)
    s_issue(0, 0)
    g_issue(2, 2)
    g_wait(1, 1)
    s_wait(0, 0)
    g_issue(3, 0)
    s_issue(1, 1)
    g_wait(2, 2)
    s_wait(1, 1)
    g_issue(4, 1)
    s_issue(2, 2)

    def pipe(p, _):
        c = 3 * p
        s_wait(c - 1, 2)
        g_issue(c + 2, 2)
        g_wait(c, 0)
        s_issue(c, 0)
        g_wait(c + 1, 1)
        s_wait(c, 0)
        g_issue(c + 3, 0)
        s_issue(c + 1, 1)
        g_wait(c + 2, 2)
        s_issue(c + 2, 2)
        s_wait(c + 1, 1)
        g_issue(c + 4, 1)
        return 0
    lax.fori_loop(1, (NCHUNK - 2) // 3, pipe, 0)

    last = NCHUNK - 1
    s_wait(last - 3, 2)
    g_wait(last - 1, 0)
    s_issue(last - 1, 0)
    g_wait(last, 1)
    s_wait(last - 1, 0)
    s_issue(last, 1)
    s_wait(last, 1)

    plsc.subcore_barrier()

    def wacc(k, _):
        r = (sid + k * NS) * CH
        pltpu.sync_copy(acc_sh.at[pl.ds(r, CH)], sums_hbm.at[cid, pl.ds(r, CH)])
        pltpu.sync_copy(cnt_sh.at[pl.ds(r, CH)], ones_v)
        pltpu.sync_copy(ones_v, cnt_hbm.at[pl.ds(cid * N + r, CH)])
        return 0
    lax.fori_loop(0, nmine, wacc, 0)


_sc_agg = pl.kernel(
    _sc_body,
    out_type=[
        jax.ShapeDtypeStruct((NC, N, D), jnp.float32),
        jax.ShapeDtypeStruct((NC * N,), jnp.float32),
    ],
    mesh=plsc.VectorSubcoreMesh(
        core_axis_name="c", subcore_axis_name="s",
        num_cores=NC, num_subcores=NS),
    scratch_types=[
        pltpu.VMEM((EP,), jnp.int32),
        pltpu.VMEM((3, CH), jnp.int32),
        pltpu.VMEM((3, CH, D), jnp.float32),
        pltpu.VMEM((CH,), jnp.float32),
        pltpu.VMEM_SHARED((N, D), jnp.float32),
        pltpu.VMEM_SHARED((N,), jnp.float32),
        pltpu.SemaphoreType.DMA,
        pltpu.SemaphoreType.DMA,
        pltpu.SemaphoreType.DMA,
        pltpu.SemaphoreType.DMA,
        pltpu.SemaphoreType.DMA,
        pltpu.SemaphoreType.DMA,
    ],
)


RBB = 2048
NPAD = 10240


def _post_body(c_ref, s_ref, n_ref, o_ref):
    c = jnp.abs(c_ref[0])
    sqrt_c = jnp.sqrt(c)
    s = s_ref[0] + s_ref[1]
    i = pl.program_id(0)
    cnt = (n_ref[0, pl.ds(i * RBB, RBB)] + n_ref[1, pl.ds(i * RBB, RBB)])[:, None]
    neigh = jnp.where(cnt > 0, s / jnp.clip(cnt, 1.0, None), 0.0)
    norm = jnp.sqrt(jnp.sum(neigh * neigh, axis=1, keepdims=True))
    norm = jnp.clip(norm, 1e-15, None)
    o_ref[...] = jnp.tanh(sqrt_c * norm) * neigh / (sqrt_c * norm)


def _post(sums, cnts, cval):
    cnts_p = jnp.concatenate(
        [cnts, jnp.zeros((NC, NPAD - N), jnp.float32)], axis=1)
    return pl.pallas_call(
        _post_body,
        grid=((N + RBB - 1) // RBB,),
        in_specs=[
            pl.BlockSpec(memory_space=pltpu.SMEM),
            pl.BlockSpec((NC, RBB, D), lambda i: (0, i, 0)),
            pl.BlockSpec((NC, NPAD), lambda i: (0, 0)),
        ],
        out_specs=pl.BlockSpec((RBB, D), lambda i: (i, 0)),
        out_shape=jax.ShapeDtypeStruct((N, D), jnp.float32),
    )(cval, sums, cnts_p)


def kernel(edge_index, node_embeddings, W, b, curvature):
    cval = jnp.abs(curvature).astype(jnp.float32)
    wt = W.T
    b2 = b.reshape(1, D)
    transformed = _pre(node_embeddings, wt, b2, cval)
    sums, cnts = _sc_agg(transformed, edge_index[0], edge_index[1])
    return _post(sums, cnts.reshape(NC, N), cval)
